# baseline (speedup 1.0000x reference)
import functools

import jax
import jax.numpy as jnp
from jax import lax
from jax.experimental import pallas as pl
from jax.experimental.pallas import tpu as pltpu
from jax.experimental.pallas import tpu_sc as plsc

VOCAB = 1000000
D = 64
S = 200
B = 4096
NC, NS = 2, 16
NW = NC * NS
ROWS_PER_W = B // NW
R = 4
CH = ROWS_PER_W // R
SPLIT = (104, 96)
LANES = 16
DC = D // LANES
UNROLL = 4

_mesh = plsc.VectorSubcoreMesh(core_axis_name="c", subcore_axis_name="s")


@functools.partial(
    pl.kernel,
    out_type=jax.ShapeDtypeStruct((B, D), jnp.float32),
    mesh=_mesh,
    compiler_params=pltpu.CompilerParams(use_tc_tiling_on_sc=False),
    scratch_types=[
        pltpu.VMEM((2, R, S), jnp.int32),
        pltpu.VMEM((2, R * S, D), jnp.float32),
        pltpu.VMEM((ROWS_PER_W, D), jnp.float32),
        pltpu.SemaphoreType.DMA,
        pltpu.SemaphoreType.DMA,
    ],
)
def _bag_kernel(x_hbm, table_hbm, out_hbm, idx_v, rows_v, out_v, sem0, sem1):
    wid = lax.axis_index("s") * NC + lax.axis_index("c")
    row0 = wid * ROWS_PER_W
    sems = (sem0, sem1)

    def start(c, slot):
        pltpu.sync_copy(x_hbm.at[pl.ds(row0 + c * R, R)], idx_v.at[slot])
        for j in range(R):
            off = 0
            for w in SPLIT:
                pltpu.async_copy(
                    table_hbm.at[idx_v.at[slot].at[j].at[pl.ds(off, w)]],
                    rows_v.at[slot, pl.ds(j * S + off, w)],
                    sems[slot],
                )
                off += w

    def wait(slot):
        pltpu.make_async_copy(
            table_hbm.at[pl.ds(0, R * S)], rows_v.at[slot], sems[slot]
        ).wait()

    def compute(c, slot):
        for j in range(R):
            base = j * S

            def rbody(i, acc, _base=base, _slot=slot):
                for u in range(UNROLL):
                    r = _base + i * UNROLL + u
                    acc = tuple(
                        acc[g] + rows_v[_slot, r, pl.ds(g * LANES, LANES)]
                        for g in range(DC)
                    )
                return acc

            zero = jnp.zeros((LANES,), jnp.float32)
            acc = lax.fori_loop(0, S // UNROLL, rbody, (zero,) * DC)
            lr = c * R + j
            for g in range(DC):
                out_v[lr, pl.ds(g * LANES, LANES)] = acc[g]

    start(0, 0)
    start(1, 1)

    def pipe(p, carry):
        c0 = 2 * p
        wait(0)
        compute(c0, 0)
        start(c0 + 2, 0)
        wait(1)
        compute(c0 + 1, 1)
        start(c0 + 3, 1)
        return carry

    lax.fori_loop(0, CH // 2 - 1, pipe, 0)

    wait(0)
    compute(CH - 2, 0)
    wait(1)
    compute(CH - 1, 1)

    pltpu.sync_copy(out_v, out_hbm.at[pl.ds(row0, ROWS_PER_W)])


VB = 2048
HB = VB // 2
LOG2_HB = HB.bit_length() - 1
G1 = (VOCAB + VB - 1) // VB


def _fmt_body(t_ref, o_ref):
    eye = jnp.eye(D, dtype=jnp.float32)
    tr = jax.lax.dot_general(
        t_ref[...], eye, (((0,), (0,)), ((), ())),
        precision=jax.lax.Precision.HIGHEST,
    )
    o_ref[:, 0:D] = tr[0:HB, :]
    o_ref[:, D : 2 * D] = tr[HB:VB, :]


_fmt_kernel = pl.pallas_call(
    _fmt_body,
    grid=(G1,),
    in_specs=[pl.BlockSpec((D, VB), lambda i: (0, i))],
    out_specs=pl.BlockSpec((VB // 2, 2 * D), lambda i: (i, 0)),
    out_shape=jax.ShapeDtypeStruct((G1 * VB // 2, 2 * D), jnp.float32),
)


def kernel(x, table):
    dense = _fmt_kernel(table.T).reshape(G1 * VB, D)
    xr = (x & ~(VB - 1)) | ((x & (HB - 1)) << 1) | ((x >> LOG2_HB) & 1)
    return _bag_kernel(xr, dense)


# speedup vs baseline: 1.6087x; 1.6087x over previous
import functools

import jax
import jax.numpy as jnp
from jax import lax
from jax.experimental import pallas as pl
from jax.experimental.pallas import tpu as pltpu
from jax.experimental.pallas import tpu_sc as plsc

VOCAB = 1000000
D = 64
S = 200
B = 4096
NC, NS = 2, 16
NW = NC * NS
ROWS_PER_W = B // NW
R = 4
CH = ROWS_PER_W // R
SPLIT = (104, 96)
LANES = 16
DC = D // LANES
UNROLL = 4

_mesh = plsc.VectorSubcoreMesh(core_axis_name="c", subcore_axis_name="s")


@functools.partial(
    pl.kernel,
    out_type=jax.ShapeDtypeStruct((B, D), jnp.float32),
    mesh=_mesh,
    compiler_params=pltpu.CompilerParams(use_tc_tiling_on_sc=False),
    scratch_types=[
        pltpu.VMEM((2, R, S), jnp.int32),
        pltpu.VMEM((2, R * S, D), jnp.float32),
        pltpu.VMEM((ROWS_PER_W, D), jnp.float32),
        pltpu.SemaphoreType.DMA,
        pltpu.SemaphoreType.DMA,
    ],
)
def _bag_kernel(x_hbm, table_hbm, out_hbm, idx_v, rows_v, out_v, sem0, sem1):
    wid = lax.axis_index("s") * NC + lax.axis_index("c")
    row0 = wid * ROWS_PER_W
    sems = (sem0, sem1)

    def start(c, slot):
        pltpu.sync_copy(x_hbm.at[pl.ds(row0 + c * R, R)], idx_v.at[slot])
        for j in range(R):
            off = 0
            for w in SPLIT:
                pltpu.async_copy(
                    table_hbm.at[idx_v.at[slot].at[j].at[pl.ds(off, w)]],
                    rows_v.at[slot, pl.ds(j * S + off, w)],
                    sems[slot],
                )
                off += w

    def wait(slot):
        pltpu.make_async_copy(
            table_hbm.at[pl.ds(0, R * S)], rows_v.at[slot], sems[slot]
        ).wait()

    def compute(c, slot):
        for j in range(R):
            base = j * S

            def rbody(i, acc, _base=base, _slot=slot):
                for u in range(UNROLL):
                    r = _base + i * UNROLL + u
                    acc = tuple(
                        acc[g] + rows_v[_slot, r, pl.ds(g * LANES, LANES)]
                        for g in range(DC)
                    )
                return acc

            zero = jnp.zeros((LANES,), jnp.float32)
            acc = lax.fori_loop(0, S // UNROLL, rbody, (zero,) * DC)
            lr = c * R + j
            for g in range(DC):
                out_v[lr, pl.ds(g * LANES, LANES)] = acc[g]

    start(0, 0)
    start(1, 1)

    def pipe(p, carry):
        c0 = 2 * p
        wait(0)
        compute(c0, 0)
        start(c0 + 2, 0)
        wait(1)
        compute(c0 + 1, 1)
        start(c0 + 3, 1)
        return carry

    lax.fori_loop(0, CH // 2 - 1, pipe, 0)

    wait(0)
    compute(CH - 2, 0)
    wait(1)
    compute(CH - 1, 1)

    pltpu.sync_copy(out_v, out_hbm.at[pl.ds(row0, ROWS_PER_W)])


VB = 4096
HB = VB // 2
LOG2_HB = HB.bit_length() - 1
G1 = (VOCAB + VB - 1) // VB


def _fmt_body(t_ref, o_ref):
    tr = t_ref[...].T
    o_ref[...] = jnp.concatenate([tr[0:HB, :], tr[HB:VB, :]], axis=1)


_fmt_kernel = pl.pallas_call(
    _fmt_body,
    grid=(G1,),
    in_specs=[pl.BlockSpec((D, VB), lambda i: (0, i))],
    out_specs=pl.BlockSpec((VB // 2, 2 * D), lambda i: (i, 0)),
    out_shape=jax.ShapeDtypeStruct((G1 * VB // 2, 2 * D), jnp.float32),
)


def kernel(x, table):
    dense = _fmt_kernel(table.T).reshape(G1 * VB, D)
    xr = (x & ~(VB - 1)) | ((x & (HB - 1)) << 1) | ((x >> LOG2_HB) & 1)
    return _bag_kernel(xr, dense)


# speedup vs baseline: 1.8938x; 1.1773x over previous
import functools

import jax
import jax.numpy as jnp
from jax import lax
from jax.experimental import pallas as pl
from jax.experimental.pallas import tpu as pltpu
from jax.experimental.pallas import tpu_sc as plsc

VOCAB = 1000000
D = 64
S = 200
B = 4096
NC, NS = 2, 16
NW = NC * NS
ROWS_PER_W = B // NW
R = 4
CH = ROWS_PER_W // R
SPLIT = (104, 96)
LANES = 16
DC = D // LANES
UNROLL = 4

_mesh = plsc.VectorSubcoreMesh(core_axis_name="c", subcore_axis_name="s")


@functools.partial(
    pl.kernel,
    out_type=jax.ShapeDtypeStruct((B, D), jnp.float32),
    mesh=_mesh,
    compiler_params=pltpu.CompilerParams(use_tc_tiling_on_sc=False),
    scratch_types=[
        pltpu.VMEM((2, R, S), jnp.int32),
        pltpu.VMEM((2, R * S, D), jnp.float32),
        pltpu.VMEM((ROWS_PER_W, D), jnp.float32),
        pltpu.SemaphoreType.DMA,
        pltpu.SemaphoreType.DMA,
    ],
)
def _bag_kernel(x_hbm, table_hbm, out_hbm, idx_v, rows_v, out_v, sem0, sem1):
    wid = lax.axis_index("s") * NC + lax.axis_index("c")
    row0 = wid * ROWS_PER_W
    sems = (sem0, sem1)

    def start(c, slot):
        pltpu.sync_copy(x_hbm.at[pl.ds(row0 + c * R, R)], idx_v.at[slot])
        for j in range(R):
            off = 0
            for w in SPLIT:
                pltpu.async_copy(
                    table_hbm.at[idx_v.at[slot].at[j].at[pl.ds(off, w)]],
                    rows_v.at[slot, pl.ds(j * S + off, w)],
                    sems[slot],
                )
                off += w

    def wait(slot):
        pltpu.make_async_copy(
            table_hbm.at[pl.ds(0, R * S)], rows_v.at[slot], sems[slot]
        ).wait()

    def compute(c, slot):
        for j in range(R):
            base = j * S

            def rbody(i, acc, _base=base, _slot=slot):
                for u in range(UNROLL):
                    r = _base + i * UNROLL + u
                    acc = tuple(
                        acc[g] + rows_v[_slot, r, pl.ds(g * LANES, LANES)]
                        for g in range(DC)
                    )
                return acc

            zero = jnp.zeros((LANES,), jnp.float32)
            acc = lax.fori_loop(0, S // UNROLL, rbody, (zero,) * DC)
            lr = c * R + j
            for g in range(DC):
                out_v[lr, pl.ds(g * LANES, LANES)] = acc[g]

    start(0, 0)
    start(1, 1)

    def pipe(p, carry):
        c0 = 2 * p
        wait(0)
        compute(c0, 0)
        start(c0 + 2, 0)
        wait(1)
        compute(c0 + 1, 1)
        start(c0 + 3, 1)
        return carry

    lax.fori_loop(0, CH // 2 - 1, pipe, 0)

    wait(0)
    compute(CH - 2, 0)
    wait(1)
    compute(CH - 1, 1)

    pltpu.sync_copy(out_v, out_hbm.at[pl.ds(row0, ROWS_PER_W)])


VB = 8192
HB = VB // 2
LOG2_HB = HB.bit_length() - 1
G1 = (VOCAB + VB - 1) // VB


def _fmt_body(t_ref, o_ref):
    tr = t_ref[...].T
    o_ref[...] = jnp.concatenate([tr[0:HB, :], tr[HB:VB, :]], axis=1)


_fmt_kernel = pl.pallas_call(
    _fmt_body,
    grid=(G1,),
    in_specs=[pl.BlockSpec((D, VB), lambda i: (0, i))],
    out_specs=pl.BlockSpec((VB // 2, 2 * D), lambda i: (i, 0)),
    out_shape=jax.ShapeDtypeStruct((G1 * VB // 2, 2 * D), jnp.float32),
)


def kernel(x, table):
    dense = _fmt_kernel(table.T).reshape(G1 * VB, D)
    xr = (x & ~(VB - 1)) | ((x & (HB - 1)) << 1) | ((x >> LOG2_HB) & 1)
    return _bag_kernel(xr, dense)


# speedup vs baseline: 2.0713x; 1.0937x over previous
import functools

import jax
import jax.numpy as jnp
from jax import lax
from jax.experimental import pallas as pl
from jax.experimental.pallas import tpu as pltpu
from jax.experimental.pallas import tpu_sc as plsc

VOCAB = 1000000
D = 64
S = 200
B = 4096
NC, NS = 2, 16
NW = NC * NS
ROWS_PER_W = B // NW
R = 4
CH = ROWS_PER_W // R
SPLIT = (104, 96)
LANES = 16
DC = D // LANES
UNROLL = 4

_mesh = plsc.VectorSubcoreMesh(core_axis_name="c", subcore_axis_name="s")


@functools.partial(
    pl.kernel,
    out_type=jax.ShapeDtypeStruct((B, D), jnp.float32),
    mesh=_mesh,
    compiler_params=pltpu.CompilerParams(use_tc_tiling_on_sc=False),
    scratch_types=[
        pltpu.VMEM((2, R, S), jnp.int32),
        pltpu.VMEM((2, R * S, D), jnp.float32),
        pltpu.VMEM((ROWS_PER_W, D), jnp.float32),
        pltpu.SemaphoreType.DMA,
        pltpu.SemaphoreType.DMA,
    ],
)
def _bag_kernel(x_hbm, table_hbm, out_hbm, idx_v, rows_v, out_v, sem0, sem1):
    wid = lax.axis_index("s") * NC + lax.axis_index("c")
    row0 = wid * ROWS_PER_W
    sems = (sem0, sem1)

    def start(c, slot):
        pltpu.sync_copy(x_hbm.at[pl.ds(row0 + c * R, R)], idx_v.at[slot])
        for j in range(R):
            off = 0
            for w in SPLIT:
                pltpu.async_copy(
                    table_hbm.at[idx_v.at[slot].at[j].at[pl.ds(off, w)]],
                    rows_v.at[slot, pl.ds(j * S + off, w)],
                    sems[slot],
                )
                off += w

    def wait(slot):
        pltpu.make_async_copy(
            table_hbm.at[pl.ds(0, R * S)], rows_v.at[slot], sems[slot]
        ).wait()

    def compute(c, slot):
        for j in range(R):
            base = j * S

            def rbody(i, acc, _base=base, _slot=slot):
                for u in range(UNROLL):
                    r = _base + i * UNROLL + u
                    acc = tuple(
                        acc[g] + rows_v[_slot, r, pl.ds(g * LANES, LANES)]
                        for g in range(DC)
                    )
                return acc

            zero = jnp.zeros((LANES,), jnp.float32)
            acc = lax.fori_loop(0, S // UNROLL, rbody, (zero,) * DC)
            lr = c * R + j
            for g in range(DC):
                out_v[lr, pl.ds(g * LANES, LANES)] = acc[g]

    start(0, 0)
    start(1, 1)

    def pipe(p, carry):
        c0 = 2 * p
        wait(0)
        compute(c0, 0)
        start(c0 + 2, 0)
        wait(1)
        compute(c0 + 1, 1)
        start(c0 + 3, 1)
        return carry

    lax.fori_loop(0, CH // 2 - 1, pipe, 0)

    wait(0)
    compute(CH - 2, 0)
    wait(1)
    compute(CH - 1, 1)

    pltpu.sync_copy(out_v, out_hbm.at[pl.ds(row0, ROWS_PER_W)])


VB = 16384
HB = VB // 2
LOG2_HB = HB.bit_length() - 1
G1 = (VOCAB + VB - 1) // VB


def _fmt_body(t_ref, o_ref):
    tr = t_ref[...].T
    o_ref[...] = jnp.concatenate([tr[0:HB, :], tr[HB:VB, :]], axis=1)


_fmt_kernel = pl.pallas_call(
    _fmt_body,
    grid=(G1,),
    in_specs=[pl.BlockSpec((D, VB), lambda i: (0, i))],
    out_specs=pl.BlockSpec((VB // 2, 2 * D), lambda i: (i, 0)),
    out_shape=jax.ShapeDtypeStruct((G1 * VB // 2, 2 * D), jnp.float32),
)


def kernel(x, table):
    dense = _fmt_kernel(table.T).reshape(G1 * VB, D)
    xr = (x & ~(VB - 1)) | ((x & (HB - 1)) << 1) | ((x >> LOG2_HB) & 1)
    return _bag_kernel(xr, dense)


# speedup vs baseline: 2.1642x; 1.0448x over previous
import functools

import jax
import jax.numpy as jnp
from jax import lax
from jax.experimental import pallas as pl
from jax.experimental.pallas import tpu as pltpu
from jax.experimental.pallas import tpu_sc as plsc

VOCAB = 1000000
D = 64
S = 200
B = 4096
NC, NS = 2, 16
NW = NC * NS
ROWS_PER_W = B // NW
R = 4
CH = ROWS_PER_W // R
SPLIT = (104, 96)
LANES = 16
DC = D // LANES
UNROLL = 4

_mesh = plsc.VectorSubcoreMesh(core_axis_name="c", subcore_axis_name="s")


@functools.partial(
    pl.kernel,
    out_type=jax.ShapeDtypeStruct((B, D), jnp.float32),
    mesh=_mesh,
    compiler_params=pltpu.CompilerParams(use_tc_tiling_on_sc=False),
    scratch_types=[
        pltpu.VMEM((2, R, S), jnp.int32),
        pltpu.VMEM((2, R * S, D), jnp.float32),
        pltpu.VMEM((ROWS_PER_W, D), jnp.float32),
        pltpu.SemaphoreType.DMA,
        pltpu.SemaphoreType.DMA,
    ],
)
def _bag_kernel(x_hbm, table_hbm, out_hbm, idx_v, rows_v, out_v, sem0, sem1):
    wid = lax.axis_index("s") * NC + lax.axis_index("c")
    row0 = wid * ROWS_PER_W
    sems = (sem0, sem1)

    def start(c, slot):
        pltpu.sync_copy(x_hbm.at[pl.ds(row0 + c * R, R)], idx_v.at[slot])
        for j in range(R):
            off = 0
            for w in SPLIT:
                pltpu.async_copy(
                    table_hbm.at[idx_v.at[slot].at[j].at[pl.ds(off, w)]],
                    rows_v.at[slot, pl.ds(j * S + off, w)],
                    sems[slot],
                )
                off += w

    def wait(slot):
        pltpu.make_async_copy(
            table_hbm.at[pl.ds(0, R * S)], rows_v.at[slot], sems[slot]
        ).wait()

    def compute(c, slot):
        for j in range(R):
            base = j * S

            def rbody(i, acc, _base=base, _slot=slot):
                for u in range(UNROLL):
                    r = _base + i * UNROLL + u
                    acc = tuple(
                        acc[g] + rows_v[_slot, r, pl.ds(g * LANES, LANES)]
                        for g in range(DC)
                    )
                return acc

            zero = jnp.zeros((LANES,), jnp.float32)
            acc = lax.fori_loop(0, S // UNROLL, rbody, (zero,) * DC)
            lr = c * R + j
            for g in range(DC):
                out_v[lr, pl.ds(g * LANES, LANES)] = acc[g]

    start(0, 0)
    start(1, 1)

    def pipe(p, carry):
        c0 = 2 * p
        wait(0)
        compute(c0, 0)
        start(c0 + 2, 0)
        wait(1)
        compute(c0 + 1, 1)
        start(c0 + 3, 1)
        return carry

    lax.fori_loop(0, CH // 2 - 1, pipe, 0)

    wait(0)
    compute(CH - 2, 0)
    wait(1)
    compute(CH - 1, 1)

    pltpu.sync_copy(out_v, out_hbm.at[pl.ds(row0, ROWS_PER_W)])


VB = 32768
HB = VB // 2
LOG2_HB = HB.bit_length() - 1
G1 = (VOCAB + VB - 1) // VB


def _fmt_body(t_ref, o_ref):
    tr = t_ref[...].T
    o_ref[...] = jnp.concatenate([tr[0:HB, :], tr[HB:VB, :]], axis=1)


_fmt_kernel = pl.pallas_call(
    _fmt_body,
    grid=(G1,),
    in_specs=[pl.BlockSpec((D, VB), lambda i: (0, i))],
    out_specs=pl.BlockSpec((VB // 2, 2 * D), lambda i: (i, 0)),
    out_shape=jax.ShapeDtypeStruct((G1 * VB // 2, 2 * D), jnp.float32),
)


def kernel(x, table):
    dense = _fmt_kernel(table.T).reshape(G1 * VB, D)
    xr = (x & ~(VB - 1)) | ((x & (HB - 1)) << 1) | ((x >> LOG2_HB) & 1)
    return _bag_kernel(xr, dense)


# speedup vs baseline: 2.2300x; 1.0304x over previous
import functools

import jax
import jax.numpy as jnp
from jax import lax
from jax.experimental import pallas as pl
from jax.experimental.pallas import tpu as pltpu
from jax.experimental.pallas import tpu_sc as plsc

VOCAB = 1000000
D = 64
S = 200
B = 4096
NC, NS = 2, 16
NW = NC * NS
ROWS_PER_W = B // NW
R = 4
CH = ROWS_PER_W // R
SPLIT = (104, 96)
LANES = 16
DC = D // LANES
UNROLL = 4

_mesh = plsc.VectorSubcoreMesh(core_axis_name="c", subcore_axis_name="s")


@functools.partial(
    pl.kernel,
    out_type=jax.ShapeDtypeStruct((B, D), jnp.float32),
    mesh=_mesh,
    compiler_params=pltpu.CompilerParams(use_tc_tiling_on_sc=False),
    scratch_types=[
        pltpu.VMEM((4, R, S), jnp.int32),
        pltpu.VMEM((2, R * S, D), jnp.float32),
        pltpu.VMEM((ROWS_PER_W, D), jnp.float32),
        pltpu.SemaphoreType.DMA,
        pltpu.SemaphoreType.DMA,
        pltpu.SemaphoreType.DMA,
        pltpu.SemaphoreType.DMA,
        pltpu.SemaphoreType.DMA,
        pltpu.SemaphoreType.DMA,
    ],
)
def _bag_kernel(
    x_hbm, table_hbm, out_hbm, idx_v, rows_v, out_v, g0, g1, i0, i1, i2, i3
):
    wid = lax.axis_index("s") * NC + lax.axis_index("c")
    row0 = wid * ROWS_PER_W
    gsem = (g0, g1)
    isem = (i0, i1, i2, i3)

    def idx_start(c, k):
        pltpu.async_copy(
            x_hbm.at[pl.ds(row0 + c * R, R)], idx_v.at[k], isem[k]
        )

    def idx_wait(k):
        pltpu.make_async_copy(
            x_hbm.at[pl.ds(0, R)], idx_v.at[k], isem[k]
        ).wait()

    def gather_start(c, k, slot):
        for j in range(R):
            off = 0
            for w in SPLIT:
                pltpu.async_copy(
                    table_hbm.at[idx_v.at[k].at[j].at[pl.ds(off, w)]],
                    rows_v.at[slot, pl.ds(j * S + off, w)],
                    gsem[slot],
                )
                off += w

    def gather_wait(slot):
        pltpu.make_async_copy(
            table_hbm.at[pl.ds(0, R * S)], rows_v.at[slot], gsem[slot]
        ).wait()

    def compute(c, slot):
        for j in range(R):
            base = j * S

            def rbody(i, acc, _base=base, _slot=slot):
                for u in range(UNROLL):
                    r = _base + i * UNROLL + u
                    acc = tuple(
                        acc[g] + rows_v[_slot, r, pl.ds(g * LANES, LANES)]
                        for g in range(DC)
                    )
                return acc

            zero = jnp.zeros((LANES,), jnp.float32)
            acc = lax.fori_loop(0, S // UNROLL, rbody, (zero,) * DC)
            lr = c * R + j
            for g in range(DC):
                out_v[lr, pl.ds(g * LANES, LANES)] = acc[g]

    for k in range(4):
        idx_start(k, k)
    idx_wait(0)
    gather_start(0, 0, 0)
    idx_wait(1)
    gather_start(1, 1, 1)

    def body(q, carry):
        c0 = 4 * q
        for k in range(4):
            c = c0 + k
            gather_wait(k % 2)
            compute(c, k % 2)
            idx_start(c + 4, k)
            idx_wait((k + 2) % 4)
            gather_start(c + 2, (k + 2) % 4, k % 2)
        return carry

    lax.fori_loop(0, CH // 4 - 1, body, 0)

    cE = CH - 4
    for k in range(4):
        c = cE + k
        gather_wait(k % 2)
        compute(c, k % 2)
        if k < 2:
            idx_wait((k + 2) % 4)
            gather_start(c + 2, (k + 2) % 4, k % 2)

    pltpu.sync_copy(out_v, out_hbm.at[pl.ds(row0, ROWS_PER_W)])


VB = 32768
HB = VB // 2
LOG2_HB = HB.bit_length() - 1
G1 = (VOCAB + VB - 1) // VB


def _fmt_body(t_ref, o_ref):
    tr = t_ref[...].T
    o_ref[...] = jnp.concatenate([tr[0:HB, :], tr[HB:VB, :]], axis=1)


_fmt_kernel = pl.pallas_call(
    _fmt_body,
    grid=(G1,),
    in_specs=[pl.BlockSpec((D, VB), lambda i: (0, i))],
    out_specs=pl.BlockSpec((VB // 2, 2 * D), lambda i: (i, 0)),
    out_shape=jax.ShapeDtypeStruct((G1 * VB // 2, 2 * D), jnp.float32),
)


def kernel(x, table):
    dense = _fmt_kernel(table.T).reshape(G1 * VB, D)
    xr = (x & ~(VB - 1)) | ((x & (HB - 1)) << 1) | ((x >> LOG2_HB) & 1)
    return _bag_kernel(xr, dense)
